# baseline (speedup 1.0000x reference)
import functools

import jax
import jax.numpy as jnp
from jax import lax
from jax.experimental import pallas as pl
from jax.experimental.pallas import tpu as pltpu
from jax.experimental.pallas import tpu_sc as plsc

D = 32
WINDOW = 64
CHUNK = 128
LANES = 16


@functools.lru_cache(maxsize=None)
def _build(n_rows: int, seq_len: int, vocab: int):
    B = n_rows * seq_len
    info = plsc.get_sparse_core_info()
    nc, ns = info.num_cores, info.num_subcores
    nw = nc * ns
    assert B % (nw * CHUNK) == 0
    chunks_pw = B // (nw * CHUNK)
    assert seq_len % WINDOW == 0 and CHUNK % WINDOW == 0

    mesh = plsc.VectorSubcoreMesh(core_axis_name="c", subcore_axis_name="s")

    @functools.partial(
        pl.kernel,
        mesh=mesh,
        out_type=jax.ShapeDtypeStruct((B, D), jnp.float32),
        scratch_types=[
            pltpu.VMEM((chunks_pw, CHUNK), jnp.int32),
            pltpu.VMEM((CHUNK, D), jnp.float32),
            pltpu.VMEM((D,), jnp.float32),
            pltpu.VMEM((D,), jnp.float32),
            pltpu.SemaphoreType.DMA,
        ],
        compiler_params=pltpu.CompilerParams(use_tc_tiling_on_sc=False),
    )
    def gather_add(idx_hbm, table_hbm, b_hbm, nb_hbm, out_hbm,
                   idx_v, rows_v, b_v, nb_v, sem):
        wid = lax.axis_index("s") * nc + lax.axis_index("c")
        pltpu.sync_copy(idx_hbm.at[pl.ds(wid * chunks_pw, chunks_pw)], idx_v)
        pltpu.sync_copy(b_hbm, b_v)
        pltpu.sync_copy(nb_hbm, nb_v)
        nb0 = nb_v[pl.ds(0, LANES)]
        nb1 = nb_v[pl.ds(LANES, LANES)]
        d0 = b_v[pl.ds(0, LANES)] - nb0
        d1 = b_v[pl.ds(LANES, LANES)] - nb1
        out_base = wid * chunks_pw * CHUNK

        def chunk_body(i, _):
            pltpu.async_copy(table_hbm.at[idx_v.at[i]], rows_v, sem).wait()

            def row_body(r, _):
                rows_v[r, pl.ds(0, LANES)] = rows_v[r, pl.ds(0, LANES)] + nb0
                rows_v[r, pl.ds(LANES, LANES)] = (
                    rows_v[r, pl.ds(LANES, LANES)] + nb1)
                return _

            lax.fori_loop(0, CHUNK, row_body, 0)
            for r in range(0, CHUNK, WINDOW):
                rows_v[r, pl.ds(0, LANES)] = rows_v[r, pl.ds(0, LANES)] + d0
                rows_v[r, pl.ds(LANES, LANES)] = (
                    rows_v[r, pl.ds(LANES, LANES)] + d1)
            pltpu.sync_copy(rows_v, out_hbm.at[pl.ds(out_base + i * CHUNK, CHUNK)])
            return _

        lax.fori_loop(0, chunks_pw, chunk_body, 0)

    return gather_add


def kernel(input, table, b_embed, nb_embed):
    n, l = input.shape
    vocab = table.shape[0]
    idx = input.reshape(-1).reshape(-1, CHUNK)
    out = _build(n, l, vocab)(idx, table, b_embed, nb_embed)
    return out.reshape(n, l, D)


# speedup vs baseline: 1.3164x; 1.3164x over previous
import functools

import jax
import jax.numpy as jnp
from jax import lax
from jax.experimental import pallas as pl
from jax.experimental.pallas import tpu as pltpu
from jax.experimental.pallas import tpu_sc as plsc

D = 32
WINDOW = 64
CHUNK = 128
LANES = 16
NBUF = 8
G = 6


@functools.lru_cache(maxsize=None)
def _build(n_rows: int, seq_len: int, vocab: int):
    B = n_rows * seq_len
    info = plsc.get_sparse_core_info()
    nc, ns = info.num_cores, info.num_subcores
    nw = nc * ns
    assert B % (nw * CHUNK) == 0
    chunks_pw = B // (nw * CHUNK)
    assert seq_len % WINDOW == 0 and CHUNK % WINDOW == 0
    assert chunks_pw % NBUF == 0 and chunks_pw > NBUF

    mesh = plsc.VectorSubcoreMesh(core_axis_name="c", subcore_axis_name="s")

    @functools.partial(
        pl.kernel,
        mesh=mesh,
        out_type=jax.ShapeDtypeStruct((B, D), jnp.float32),
        scratch_types=[
            pltpu.VMEM((chunks_pw, CHUNK), jnp.int32),
            pltpu.VMEM((NBUF, CHUNK, D), jnp.float32),
            pltpu.VMEM((D,), jnp.float32),
            pltpu.VMEM((D,), jnp.float32),
            [pltpu.SemaphoreType.DMA] * NBUF,
            [pltpu.SemaphoreType.DMA] * NBUF,
        ],
        compiler_params=pltpu.CompilerParams(use_tc_tiling_on_sc=False),
    )
    def gather_add(idx_hbm, table_hbm, b_hbm, nb_hbm, out_hbm,
                   idx_v, rows_v, b_v, nb_v, gsem, wsem):
        wid = lax.axis_index("s") * nc + lax.axis_index("c")
        pltpu.sync_copy(idx_hbm.at[pl.ds(wid * chunks_pw, chunks_pw)], idx_v)
        pltpu.sync_copy(b_hbm, b_v)
        pltpu.sync_copy(nb_hbm, nb_v)
        nb0 = nb_v[pl.ds(0, LANES)]
        nb1 = nb_v[pl.ds(LANES, LANES)]
        d0 = b_v[pl.ds(0, LANES)] - nb0
        d1 = b_v[pl.ds(LANES, LANES)] - nb1
        out_base = wid * chunks_pw * CHUNK

        def gather(chunk, slot):
            return pltpu.make_async_copy(
                table_hbm.at[idx_v.at[chunk]], rows_v.at[slot], gsem[slot])

        def write(chunk, slot):
            return pltpu.make_async_copy(
                rows_v.at[slot],
                out_hbm.at[pl.ds(out_base + chunk * CHUNK, CHUNK)],
                wsem[slot])

        for b in range(G):
            gather(b, b).start()

        @pl.loop(0, chunks_pw, step=NBUF)
        def _(i0):
            for b in range(NBUF):
                i = i0 + b
                sp = (b + G) % NBUF
                p = i + G

                @pl.when(p < chunks_pw)
                def _():
                    @pl.when(p - NBUF >= 0)
                    def _():
                        write(p - NBUF, sp).wait()
                    gather(p, sp).start()

                gather(i, b).wait()

                @plsc.parallel_loop(0, CHUNK, 1, unroll=8)
                def _(r):
                    rows_v[b, r, pl.ds(0, LANES)] = (
                        rows_v[b, r, pl.ds(0, LANES)] + nb0)
                    rows_v[b, r, pl.ds(LANES, LANES)] = (
                        rows_v[b, r, pl.ds(LANES, LANES)] + nb1)

                for r in range(0, CHUNK, WINDOW):
                    rows_v[b, r, pl.ds(0, LANES)] = (
                        rows_v[b, r, pl.ds(0, LANES)] + d0)
                    rows_v[b, r, pl.ds(LANES, LANES)] = (
                        rows_v[b, r, pl.ds(LANES, LANES)] + d1)
                write(i, b).start()

        for b in range(NBUF):
            write(chunks_pw - NBUF + b, (chunks_pw - NBUF + b) % NBUF).wait()

    return gather_add


def kernel(input, table, b_embed, nb_embed):
    n, l = input.shape
    vocab = table.shape[0]
    idx = input.reshape(-1).reshape(-1, CHUNK)
    out = _build(n, l, vocab)(idx, table, b_embed, nb_embed)
    return out.reshape(n, l, D)


# speedup vs baseline: 1.3363x; 1.0151x over previous
import functools

import jax
import jax.numpy as jnp
from jax import lax
from jax.experimental import pallas as pl
from jax.experimental.pallas import tpu as pltpu
from jax.experimental.pallas import tpu_sc as plsc

D = 32
WINDOW = 64
CHUNK = 128
LANES = 16
G = 6


@functools.lru_cache(maxsize=None)
def _build(n_rows: int, seq_len: int, vocab: int):
    B = n_rows * seq_len
    info = plsc.get_sparse_core_info()
    nc, ns = info.num_cores, info.num_subcores
    nw = nc * ns
    cps = seq_len // CHUNK
    slices_pw = n_rows // nw
    chunks_pw = slices_pw * cps
    assert B == nw * chunks_pw * CHUNK
    assert seq_len % WINDOW == 0 and CHUNK % WINDOW == 0
    assert cps == 8 and slices_pw % 2 == 0
    sl_bytes = seq_len * D

    mesh = plsc.VectorSubcoreMesh(core_axis_name="c", subcore_axis_name="s")

    @functools.partial(
        pl.kernel,
        mesh=mesh,
        out_type=jax.ShapeDtypeStruct((n_rows, sl_bytes), jnp.float32),
        scratch_types=[
            pltpu.VMEM((2, cps, CHUNK), jnp.int32),
            pltpu.VMEM((cps, CHUNK, D), jnp.float32),
            pltpu.VMEM((2, sl_bytes), jnp.float32),
            pltpu.VMEM((D,), jnp.float32),
            pltpu.VMEM((D,), jnp.float32),
            [pltpu.SemaphoreType.DMA] * cps,
            [pltpu.SemaphoreType.DMA] * 2,
            [pltpu.SemaphoreType.DMA] * 2,
        ],
        compiler_params=pltpu.CompilerParams(
            use_tc_tiling_on_sc=False, needs_layout_passes=False),
    )
    def gather_add(idx_hbm, table_hbm, b_hbm, nb_hbm, out_hbm,
                   idx_v, rows_v, block_v, b_v, nb_v, gsem, bsem, isem):
        wid = lax.axis_index("s") * nc + lax.axis_index("c")
        cbase = wid * chunks_pw
        nbase = wid * slices_pw
        pltpu.sync_copy(b_hbm, b_v)
        pltpu.sync_copy(nb_hbm, nb_v)
        nb0 = nb_v[pl.ds(0, LANES)]
        nb1 = nb_v[pl.ds(LANES, LANES)]
        d0 = b_v[pl.ds(0, LANES)] - nb0
        d1 = b_v[pl.ds(LANES, LANES)] - nb1
        lane = lax.broadcasted_iota(jnp.int32, (LANES,), 0)
        off0 = (lane // 8) * (8 * seq_len) + (lane % 8) * CHUNK
        off1 = off0 + 2 * (8 * seq_len)

        def idx_load(sl, par):
            return pltpu.make_async_copy(
                idx_hbm.at[pl.ds(cbase + sl * cps, cps)], idx_v.at[par],
                isem[par])

        def gather(sl, c, par, slot):
            return pltpu.make_async_copy(
                table_hbm.at[idx_v.at[par, c]], rows_v.at[slot], gsem[slot])

        def block_write(sl, par):
            return pltpu.make_async_copy(
                block_v.at[par], out_hbm.at[nbase + sl], bsem[par])

        idx_load(0, 0).start()
        idx_load(0, 0).wait()
        idx_load(1, 1).start()
        for c in range(G):
            gather(0, c, 0, c).start()

        @pl.loop(0, slices_pw, step=2)
        def _(k0):
            for pp in range(2):
                k = k0 + pp
                for b in range(cps):
                    if b == 0:
                        @pl.when(k >= 2)
                        def _():
                            block_write(k - 2, pp).wait()
                    if b == 2:
                        @pl.when(k + 1 < slices_pw)
                        def _():
                            idx_load(k + 1, (pp + 1) % 2).wait()

                        @pl.when(k + 2 < slices_pw)
                        def _():
                            idx_load(k + 2, pp).start()
                    cf = b + G
                    par_f = (pp + cf // cps) % 2

                    @pl.when(k * cps + cf < chunks_pw)
                    def _():
                        gather(k + cf // cps, cf % cps, par_f, cf % cps).start()

                    gather(k, b, pp, b).wait()

                    blk = block_v.at[pp]
                    base_off0 = off0 + b * seq_len
                    base_off1 = off1 + b * seq_len

                    @plsc.parallel_loop(0, CHUNK, 1, unroll=8)
                    def _(r):
                        v0 = rows_v[b, r, pl.ds(0, LANES)] + nb0
                        v1 = rows_v[b, r, pl.ds(LANES, LANES)] + nb1
                        plsc.store_scatter(blk, [base_off0 + r], v0)
                        plsc.store_scatter(blk, [base_off1 + r], v1)

                    for r in range(0, CHUNK, WINDOW):
                        i0r = base_off0 + r
                        i1r = base_off1 + r
                        w0 = plsc.load_gather(blk, [i0r]) + d0
                        w1 = plsc.load_gather(blk, [i1r]) + d1
                        plsc.store_scatter(blk, [i0r], w0)
                        plsc.store_scatter(blk, [i1r], w1)

                    if b == cps - 1:
                        block_write(k, pp).start()

        for k in (slices_pw - 2, slices_pw - 1):
            block_write(k, k % 2).wait()

    def kernel_fn(input, table, b_embed, nb_embed):
        n, l = input.shape
        idx = input.reshape(-1).reshape(-1, CHUNK)
        out2 = gather_add(idx, table, b_embed, nb_embed)
        out5 = out2.reshape(n, D // 8, l // CHUNK, 8, CHUNK)
        return jnp.transpose(out5, (0, 2, 4, 1, 3)).reshape(n, l, D)

    return kernel_fn


def kernel(input, table, b_embed, nb_embed):
    n, l = input.shape
    return _build(n, l, table.shape[0])(input, table, b_embed, nb_embed)
